# Initial kernel scaffold; baseline (speedup 1.0000x reference)
import jax
import jax.numpy as jnp
from jax.experimental import pallas as pl


def kernel(x, edge_index, edge_weight, batch, W1, b1, W2, b2, Ws, bs, W3, b3, W4, b4, Wl, bl):
    raise NotImplementedError("write your pallas kernel here")


# speedup vs baseline: 5.5249x; 5.5249x over previous
import functools

import jax
import jax.numpy as jnp
from jax import lax
from jax.experimental import pallas as pl
from jax.experimental.pallas import tpu as pltpu
from jax.experimental.pallas import tpu_sc as plsc

NC = 2
NS = 16
G = 8

F32 = jnp.float32
I32 = jnp.int32


def _sc_agg_kernel(NP, ER, n_iters):
  rows_per_tile = NP // NS

  mesh = plsc.VectorSubcoreMesh(
      core_axis_name="c", subcore_axis_name="s", num_cores=NC, num_subcores=NS)

  @functools.partial(
      pl.kernel,
      out_type=(jax.ShapeDtypeStruct((NP, 16), F32),
                jax.ShapeDtypeStruct((NP, 16), F32)),
      mesh=mesh,
      scratch_types=[
          pltpu.VMEM((G, 128), I32),
          pltpu.VMEM((G, 128), I32),
          pltpu.VMEM((G, 128), F32),
          pltpu.VMEM((G * 128, 16), F32),
          pltpu.VMEM_SHARED((NP, 16), F32),
          pltpu.SemaphoreType.DMA,
      ],
      compiler_params=pltpu.CompilerParams(use_tc_tiling_on_sc=False),
  )
  def k(x16_h, src_h, dst_h, w_h, out0_h, out1_h,
        idx_s, idx_d, w_v, rows_v, acc_sh, sem):
    c = lax.axis_index("c")
    s = lax.axis_index("s")

    @plsc.parallel_loop(0, G * 128, 1, unroll=8)
    def _(i):
      rows_v[i, :] = jnp.zeros((16,), F32)

    r0 = s * rows_per_tile
    done = 0
    while done < rows_per_tile:
      n = min(G * 128, rows_per_tile - done)
      pltpu.sync_copy(rows_v.at[pl.ds(0, n)], acc_sh.at[pl.ds(r0 + done, n)])
      done += n
    plsc.subcore_barrier()

    @pl.loop(0, n_iters)
    def _(t):
      rb = c * (ER // NC) + s * (ER // NC // NS) + t * G
      pltpu.sync_copy(src_h.at[pl.ds(rb, G)], idx_s)
      pltpu.sync_copy(dst_h.at[pl.ds(rb, G)], idx_d)
      pltpu.sync_copy(w_h.at[pl.ds(rb, G)], w_v)
      descs = [
          pltpu.async_copy(x16_h.at[idx_s.at[j]],
                           rows_v.at[pl.ds(j * 128, 128)], sem)
          for j in range(G)
      ]
      for d in descs:
        d.wait()

      @plsc.parallel_loop(0, G * 8, 1)
      def _(g):
        w16 = w_v[g >> 3, pl.ds((g & 7) * 16, 16)]
        lane7 = lax.iota(I32, 16) == 7
        for m in range(16):
          r = g * 16 + m
          rows_v[r, :] = jnp.where(lane7, w16[m], rows_v[r, :])

      for j in range(G):
        pltpu.sync_copy(rows_v.at[pl.ds(j * 128, 128)],
                        acc_sh.at[idx_d.at[j]], add=True)

    plsc.subcore_barrier()

    @pl.when(c == 0)
    def _():
      pltpu.sync_copy(acc_sh.at[pl.ds(r0, rows_per_tile)],
                      out0_h.at[pl.ds(r0, rows_per_tile)])

    @pl.when(c == 1)
    def _():
      pltpu.sync_copy(acc_sh.at[pl.ds(r0, rows_per_tile)],
                      out1_h.at[pl.ds(r0, rows_per_tile)])

  return k


def _sc_pool_kernel(NP, ER, n_iters):
  rows_per_tile = NP // NS

  mesh = plsc.VectorSubcoreMesh(
      core_axis_name="c", subcore_axis_name="s", num_cores=NC, num_subcores=NS)

  @functools.partial(
      pl.kernel,
      out_type=tuple(jax.ShapeDtypeStruct((NP, 16), F32) for _ in range(4)),
      mesh=mesh,
      scratch_types=[
          pltpu.VMEM((G, 128), I32),
          pltpu.VMEM((G, 128), I32),
          pltpu.VMEM((G, 128), F32),
          pltpu.VMEM((G * 128, 16), F32),
          pltpu.VMEM_SHARED((NP, 16), F32),
          pltpu.SemaphoreType.DMA,
      ],
      compiler_params=pltpu.CompilerParams(use_tc_tiling_on_sc=False),
  )
  def k(s0_h, s1_h, s2_h, s3_h, src_h, dst_h, w_h, t0_h, t1_h, t2_h, t3_h,
        idx_s, idx_d, w_v, rows_v, acc_sh, sem):
    c = lax.axis_index("c")
    s = lax.axis_index("s")
    r0 = s * rows_per_tile

    def one_chunk(tab_h, out_h):
      @plsc.parallel_loop(0, G * 128, 1, unroll=8)
      def _(i):
        rows_v[i, :] = jnp.zeros((16,), F32)

      done = 0
      while done < rows_per_tile:
        n = min(G * 128, rows_per_tile - done)
        pltpu.sync_copy(rows_v.at[pl.ds(0, n)], acc_sh.at[pl.ds(r0 + done, n)])
        done += n
      plsc.subcore_barrier()

      @pl.loop(0, n_iters)
      def _(t):
        rb = s * (ER // NS) + t * G
        pltpu.sync_copy(src_h.at[pl.ds(rb, G)], idx_s)
        pltpu.sync_copy(dst_h.at[pl.ds(rb, G)], idx_d)
        pltpu.sync_copy(w_h.at[pl.ds(rb, G)], w_v)
        descs = [
            pltpu.async_copy(tab_h.at[idx_s.at[j]],
                             rows_v.at[pl.ds(j * 128, 128)], sem)
            for j in range(G)
        ]
        for d in descs:
          d.wait()

        @plsc.parallel_loop(0, G * 8, 1)
        def _(g):
          w16 = w_v[g >> 3, pl.ds((g & 7) * 16, 16)]
          for m in range(16):
            r = g * 16 + m
            rows_v[r, :] = rows_v[r, :] * w16[m]

        for j in range(G):
          pltpu.sync_copy(rows_v.at[pl.ds(j * 128, 128)],
                          acc_sh.at[idx_d.at[j]], add=True)

      plsc.subcore_barrier()
      pltpu.sync_copy(acc_sh.at[pl.ds(r0, rows_per_tile)],
                      out_h.at[pl.ds(r0, rows_per_tile)])
      plsc.subcore_barrier()

    @pl.when(c == 0)
    def _():
      one_chunk(s0_h, t0_h)
      one_chunk(s1_h, t1_h)

    @pl.when(c == 1)
    def _():
      one_chunk(s2_h, t2_h)
      one_chunk(s3_h, t3_h)

  return k


def _tc1_body(N, BN, x_ref, a0_ref, a1_ref, w1_ref, b1_ref, w2_ref, b2_ref,
              ws_ref, bs_ref, slo_ref, s1_ref, s2_ref, shi_ref,
              xp_ref, sts_ref, trd_ref):
  i = pl.program_id(0)
  A = a0_ref[...] + a1_ref[...]
  hin = x_ref[...] + A
  h1 = jnp.maximum(
      jnp.dot(hin, w1_ref[...], preferred_element_type=F32) + b1_ref[...], 0.0)
  h = jnp.dot(h1, w2_ref[...], preferred_element_type=F32) + b2_ref[...]
  logits = jnp.dot(h, ws_ref[...], preferred_element_type=F32) + bs_ref[...]
  m = jnp.max(logits, axis=-1, keepdims=True)
  e = jnp.exp(logits - m)
  S = e / jnp.sum(e, axis=-1, keepdims=True)
  rows = i * BN + lax.broadcasted_iota(I32, (BN, 1), 0)
  S = jnp.where(rows < N, S, 0.0)
  slo_ref[...] = S[:, 0:16]
  s1_ref[...] = S[:, 16:32]
  s2_ref[...] = S[:, 32:48]
  shi_ref[...] = S[:, 48:64]
  deg = A[:, 7:8]

  @pl.when(i == 0)
  def _():
    xp_ref[...] = jnp.zeros_like(xp_ref)
    sts_ref[...] = jnp.zeros_like(sts_ref)
    trd_ref[...] = jnp.zeros_like(trd_ref)

  cdims = (((0,), (0,)), ((), ()))
  xp_ref[...] += lax.dot_general(S, h, cdims, preferred_element_type=F32)
  sts_ref[...] += lax.dot_general(S, S, cdims, preferred_element_type=F32)
  trd_ref[...] += jnp.sum(deg * (S * S)).reshape(1, 1)


def _tc2_body(K, n_blocks, s0_ref, s1_ref, s2_ref, s3_ref,
              t0_ref, t1_ref, t2_ref, t3_ref, xp_ref, sts_ref,
              trd_ref, w3_ref, b3_ref, w4_ref, b4_ref, wl_ref, bl_ref,
              logp_ref, aux_ref, adj_acc):
  i = pl.program_id(0)

  @pl.when(i == 0)
  def _():
    adj_acc[...] = jnp.zeros_like(adj_acc)

  S64 = jnp.concatenate(
      [s0_ref[...], s1_ref[...], s2_ref[...], s3_ref[...]], axis=1)
  T64 = jnp.concatenate(
      [t0_ref[...], t1_ref[...], t2_ref[...], t3_ref[...]], axis=1)
  cdims = (((0,), (0,)), ((), ()))
  adj_acc[...] += lax.dot_general(T64, S64, cdims, preferred_element_type=F32)

  @pl.when(i == n_blocks - 1)
  def _():
    adj = adj_acc[...]
    ii = lax.broadcasted_iota(I32, (64, 64), 0)
    jj = lax.broadcasted_iota(I32, (64, 64), 1)
    diag = ii == jj
    cut_num = jnp.sum(jnp.where(diag, adj, 0.0))
    cut_loss = -cut_num / (trd_ref[0, 0] + 1e-9)
    sts = sts_ref[...]
    nrm = jnp.sqrt(jnp.sum(sts * sts))
    eye = jnp.where(diag & (ii < K), 1.0 / jnp.sqrt(jnp.float32(K)), 0.0)
    dmat = sts / (nrm + 1e-9) - eye
    ortho = jnp.sqrt(jnp.sum(dmat * dmat))
    aux_ref[...] = (cut_loss + ortho).reshape(1, 1)

    xp = xp_ref[...]
    hg = xp + jnp.dot(adj, xp, preferred_element_type=F32)
    hg = jnp.maximum(
        jnp.dot(hg, w3_ref[...], preferred_element_type=F32) + b3_ref[...], 0.0)
    hg = jnp.dot(hg, w4_ref[...], preferred_element_type=F32) + b4_ref[...]
    ri = lax.broadcasted_iota(I32, (64, 1), 0)
    g = jnp.sum(jnp.where(ri < K, hg, 0.0), axis=0, keepdims=True) / jnp.float32(K)
    out = jnp.dot(g, wl_ref[...], preferred_element_type=F32) + bl_ref[...]
    m = jnp.max(out, axis=-1, keepdims=True)
    lse = m + jnp.log(jnp.sum(jnp.exp(out - m), axis=-1, keepdims=True))
    logp_ref[...] = out - lse


def kernel(x, edge_index, edge_weight, batch, W1, b1, W2, b2, Ws, bs,
           W3, b3, W4, b4, Wl, bl):
  N, F = x.shape
  E = edge_index.shape[1]
  H = W1.shape[1]
  K = Ws.shape[1]
  C = Wl.shape[1]

  NP = ((N + 1 + 127) // 128) * 128
  EPQ = 128 * NS * G * NC
  EP = ((E + EPQ - 1) // EPQ) * EPQ
  ER = EP // 128

  x16 = jnp.pad(x, ((0, NP - N), (0, 16 - F)))
  pad_e = EP - E
  srcp = jnp.concatenate(
      [edge_index[0], jnp.full((pad_e,), N, I32)]).reshape(ER, 128)
  dstp = jnp.concatenate(
      [edge_index[1], jnp.full((pad_e,), N, I32)]).reshape(ER, 128)
  wp = jnp.concatenate(
      [edge_weight, jnp.zeros((pad_e,), F32)]).reshape(ER, 128)

  a0, a1 = _sc_agg_kernel(NP, ER, ER // (NC * NS * G))(x16, srcp, dstp, wp)

  W1p = jnp.pad(W1, ((0, 16 - F), (0, 0)))
  Wsp = jnp.pad(Ws, ((0, 0), (0, 64 - K)))
  bsp = jnp.concatenate([bs, jnp.full((64 - K,), -1e30, F32)]).reshape(1, 64)

  BN = NP // 16
  n_blocks = NP // BN
  grid = (n_blocks,)
  s0, s1, s2, s3, xp64, sts, trd = pl.pallas_call(
      functools.partial(_tc1_body, N, BN),
      grid=grid,
      in_specs=[
          pl.BlockSpec((BN, 16), lambda i: (i, 0)),
          pl.BlockSpec((BN, 16), lambda i: (i, 0)),
          pl.BlockSpec((BN, 16), lambda i: (i, 0)),
          pl.BlockSpec((16, 64), lambda i: (0, 0)),
          pl.BlockSpec((1, 64), lambda i: (0, 0)),
          pl.BlockSpec((64, 64), lambda i: (0, 0)),
          pl.BlockSpec((1, 64), lambda i: (0, 0)),
          pl.BlockSpec((64, 64), lambda i: (0, 0)),
          pl.BlockSpec((1, 64), lambda i: (0, 0)),
      ],
      out_specs=[
          pl.BlockSpec((BN, 16), lambda i: (i, 0)),
          pl.BlockSpec((BN, 16), lambda i: (i, 0)),
          pl.BlockSpec((BN, 16), lambda i: (i, 0)),
          pl.BlockSpec((BN, 16), lambda i: (i, 0)),
          pl.BlockSpec((64, 64), lambda i: (0, 0)),
          pl.BlockSpec((64, 64), lambda i: (0, 0)),
          pl.BlockSpec((1, 1), lambda i: (0, 0)),
      ],
      out_shape=[
          jax.ShapeDtypeStruct((NP, 16), F32),
          jax.ShapeDtypeStruct((NP, 16), F32),
          jax.ShapeDtypeStruct((NP, 16), F32),
          jax.ShapeDtypeStruct((NP, 16), F32),
          jax.ShapeDtypeStruct((64, 64), F32),
          jax.ShapeDtypeStruct((64, 64), F32),
          jax.ShapeDtypeStruct((1, 1), F32),
      ],
  )(x16, a0, a1, W1p, b1.reshape(1, H), W2, b2.reshape(1, H),
    Wsp, bsp)

  t0, t1, t2, t3 = _sc_pool_kernel(NP, ER, ER // (NS * G))(
      s0, s1, s2, s3, srcp, dstp, wp)

  Wl128 = jnp.pad(Wl, ((0, 0), (0, 128 - C)))
  bl128 = jnp.concatenate([bl, jnp.full((128 - C,), -1e30, F32)]).reshape(1, 128)

  logp128, aux = pl.pallas_call(
      functools.partial(_tc2_body, K, n_blocks),
      grid=grid,
      in_specs=[
          pl.BlockSpec((BN, 16), lambda i: (i, 0)),
          pl.BlockSpec((BN, 16), lambda i: (i, 0)),
          pl.BlockSpec((BN, 16), lambda i: (i, 0)),
          pl.BlockSpec((BN, 16), lambda i: (i, 0)),
          pl.BlockSpec((BN, 16), lambda i: (i, 0)),
          pl.BlockSpec((BN, 16), lambda i: (i, 0)),
          pl.BlockSpec((BN, 16), lambda i: (i, 0)),
          pl.BlockSpec((BN, 16), lambda i: (i, 0)),
          pl.BlockSpec((64, 64), lambda i: (0, 0)),
          pl.BlockSpec((64, 64), lambda i: (0, 0)),
          pl.BlockSpec((1, 1), lambda i: (0, 0)),
          pl.BlockSpec((64, 64), lambda i: (0, 0)),
          pl.BlockSpec((1, 64), lambda i: (0, 0)),
          pl.BlockSpec((64, 64), lambda i: (0, 0)),
          pl.BlockSpec((1, 64), lambda i: (0, 0)),
          pl.BlockSpec((64, 128), lambda i: (0, 0)),
          pl.BlockSpec((1, 128), lambda i: (0, 0)),
      ],
      out_specs=[
          pl.BlockSpec((1, 128), lambda i: (0, 0)),
          pl.BlockSpec((1, 1), lambda i: (0, 0)),
      ],
      out_shape=[
          jax.ShapeDtypeStruct((1, 128), F32),
          jax.ShapeDtypeStruct((1, 1), F32),
      ],
      scratch_shapes=[pltpu.VMEM((64, 64), F32)],
  )(s0, s1, s2, s3, t0, t1, t2, t3, xp64, sts, trd, W3, b3.reshape(1, H),
    W4, b4.reshape(1, H), Wl128, bl128)

  return (logp128[0:1, 0:C], aux[0, 0])


# speedup vs baseline: 6.6190x; 1.1980x over previous
import functools

import jax
import jax.numpy as jnp
from jax import lax
from jax.experimental import pallas as pl
from jax.experimental.pallas import tpu as pltpu
from jax.experimental.pallas import tpu_sc as plsc

NC = 2
NS = 16
G = 10

F32 = jnp.float32
I32 = jnp.int32


def _edge_pipeline(tab_h, src_h, dst_h, w_h, acc_sh, bufs, base, n_iters,
                   transform):
  (idx_s, idx_d, w_v, rows_v, gsem, ssem) = bufs
  npairs = n_iters // 2

  def stage_and_fire(t, b):
    rb = base + t * G
    pltpu.sync_copy(src_h.at[pl.ds(rb, G)], idx_s[b])
    pltpu.sync_copy(dst_h.at[pl.ds(rb, G)], idx_d[b])
    pltpu.sync_copy(w_h.at[pl.ds(rb, G)], w_v[b])
    for j in range(G):
      pltpu.async_copy(tab_h.at[idx_s[b].at[j]],
                       rows_v[b].at[pl.ds(j * 128, 128)], gsem[b])

  def wait_gathers(b):
    for j in range(G):
      pltpu.make_async_copy(tab_h.at[idx_s[b].at[j]],
                            rows_v[b].at[pl.ds(j * 128, 128)], gsem[b]).wait()

  def fire_scatters(b):
    for j in range(G):
      pltpu.async_copy(rows_v[b].at[pl.ds(j * 128, 128)],
                       acc_sh.at[idx_d[b].at[j]], ssem[b], add=True)

  def wait_scatters(b):
    for j in range(G):
      pltpu.make_async_copy(tab_h.at[idx_s[b].at[j]],
                            rows_v[b].at[pl.ds(j * 128, 128)], ssem[b]).wait()

  def sync_scatters(b):
    for j in range(G):
      pltpu.sync_copy(rows_v[b].at[pl.ds(j * 128, 128)],
                      acc_sh.at[idx_d[b].at[j]], add=True)

  stage_and_fire(0, 0)

  @pl.loop(0, npairs)
  def _(t2):
    wait_gathers(0)

    @pl.when(t2 > 0)
    def _():
      wait_scatters(1)

    stage_and_fire(2 * t2 + 1, 1)
    transform(0)
    fire_scatters(0)
    wait_gathers(1)
    wait_scatters(0)

    @pl.when(t2 < npairs - 1)
    def _():
      stage_and_fire(2 * t2 + 2, 0)

    transform(1)

    @pl.when(t2 < npairs - 1)
    def _():
      fire_scatters(1)

    @pl.when(t2 == npairs - 1)
    def _():
      sync_scatters(1)


def _zero_acc_slice(rows_v0, acc_sh, r0, rows_per_tile, width):
  @plsc.parallel_loop(0, G * 128, 1, unroll=8)
  def _(i):
    for h in range(width // 16):
      rows_v0[i, pl.ds(h * 16, 16)] = jnp.zeros((16,), F32)

  done = 0
  while done < rows_per_tile:
    n = min(G * 128, rows_per_tile - done)
    pltpu.sync_copy(rows_v0.at[pl.ds(0, n)], acc_sh.at[pl.ds(r0 + done, n)])
    done += n


def _sc_scratch(width):
  return [
      pltpu.VMEM((G, 128), I32), pltpu.VMEM((G, 128), I32),
      pltpu.VMEM((G, 128), I32), pltpu.VMEM((G, 128), I32),
      pltpu.VMEM((G, 128), F32), pltpu.VMEM((G, 128), F32),
      pltpu.VMEM((G * 128, width), F32),
      pltpu.VMEM((G * 128, width), F32),
      pltpu.SemaphoreType.DMA, pltpu.SemaphoreType.DMA,
      pltpu.SemaphoreType.DMA, pltpu.SemaphoreType.DMA,
  ]


def _sc_agg_kernel(NP, ER, n_iters):
  rows_per_tile = NP // NS

  mesh = plsc.VectorSubcoreMesh(
      core_axis_name="c", subcore_axis_name="s", num_cores=NC, num_subcores=NS)

  @functools.partial(
      pl.kernel,
      out_type=(jax.ShapeDtypeStruct((NP, 16), F32),
                jax.ShapeDtypeStruct((NP, 16), F32)),
      mesh=mesh,
      scratch_types=_sc_scratch(16)
      + [pltpu.VMEM_SHARED((NP, 16), F32)],
      compiler_params=pltpu.CompilerParams(use_tc_tiling_on_sc=False),
  )
  def k(x16_h, src_h, dst_h, w_h, out0_h, out1_h,
        is0, is1, id0, id1, wv0, wv1, rv0, rv1, gs0, gs1, ss0, ss1, acc_sh):
    c = lax.axis_index("c")
    s = lax.axis_index("s")
    r0 = s * rows_per_tile
    bufs = ((is0, is1), (id0, id1), (wv0, wv1), (rv0, rv1),
            (gs0, gs1), (ss0, ss1))

    _zero_acc_slice(rv0, acc_sh, r0, rows_per_tile, 16)
    plsc.subcore_barrier()

    def plant_w(b):
      rows_v, w_v = (rv0, rv1)[b], (wv0, wv1)[b]

      @plsc.parallel_loop(0, G * 8, 1, unroll=2)
      def _(g):
        w16 = w_v[g >> 3, pl.ds((g & 7) * 16, 16)]
        lane7 = lax.iota(I32, 16) == 7
        for m in range(16):
          r = g * 16 + m
          rows_v[r, :] = jnp.where(lane7, w16[m], rows_v[r, :])

    base = c * (ER // NC) + s * (ER // NC // NS)
    _edge_pipeline(x16_h, src_h, dst_h, w_h, acc_sh, bufs, base, n_iters,
                   plant_w)
    plsc.subcore_barrier()

    @pl.when(c == 0)
    def _():
      pltpu.sync_copy(acc_sh.at[pl.ds(r0, rows_per_tile)],
                      out0_h.at[pl.ds(r0, rows_per_tile)])

    @pl.when(c == 1)
    def _():
      pltpu.sync_copy(acc_sh.at[pl.ds(r0, rows_per_tile)],
                      out1_h.at[pl.ds(r0, rows_per_tile)])

  return k


def _sc_pool_kernel(NP, ER, n_iters):
  rows_per_tile = NP // NS

  mesh = plsc.VectorSubcoreMesh(
      core_axis_name="c", subcore_axis_name="s", num_cores=NC, num_subcores=NS)

  @functools.partial(
      pl.kernel,
      out_type=tuple(jax.ShapeDtypeStruct((NP, 16), F32) for _ in range(4)),
      mesh=mesh,
      scratch_types=_sc_scratch(16)
      + [pltpu.VMEM_SHARED((NP, 16), F32)],
      compiler_params=pltpu.CompilerParams(use_tc_tiling_on_sc=False),
  )
  def k(s0_h, s1_h, s2_h, s3_h, src_h, dst_h, w_h, t0_h, t1_h, t2_h, t3_h,
        is0, is1, id0, id1, wv0, wv1, rv0, rv1, gs0, gs1, ss0, ss1, acc_sh):
    c = lax.axis_index("c")
    s = lax.axis_index("s")
    r0 = s * rows_per_tile
    bufs = ((is0, is1), (id0, id1), (wv0, wv1), (rv0, rv1),
            (gs0, gs1), (ss0, ss1))

    def scale_w(b):
      rows_v, w_v = (rv0, rv1)[b], (wv0, wv1)[b]

      @plsc.parallel_loop(0, G * 8, 1, unroll=2)
      def _(g):
        w16 = w_v[g >> 3, pl.ds((g & 7) * 16, 16)]
        for m in range(16):
          r = g * 16 + m
          rows_v[r, :] = rows_v[r, :] * w16[m]

    def one_chunk(tab_h, out_h):
      _zero_acc_slice(rv0, acc_sh, r0, rows_per_tile, 16)
      plsc.subcore_barrier()
      base = s * (ER // NS)
      _edge_pipeline(tab_h, src_h, dst_h, w_h, acc_sh, bufs, base, n_iters,
                     scale_w)
      plsc.subcore_barrier()
      pltpu.sync_copy(acc_sh.at[pl.ds(r0, rows_per_tile)],
                      out_h.at[pl.ds(r0, rows_per_tile)])
      plsc.subcore_barrier()

    @pl.when(c == 0)
    def _():
      one_chunk(s0_h, t0_h)
      one_chunk(s1_h, t1_h)

    @pl.when(c == 1)
    def _():
      one_chunk(s2_h, t2_h)
      one_chunk(s3_h, t3_h)

  return k


def _tc1_body(N, BN, x_ref, a0_ref, a1_ref, w1_ref, b1_ref, w2_ref, b2_ref,
              ws_ref, bs_ref, slo_ref, s1_ref, s2_ref, shi_ref,
              xp_ref, sts_ref, trd_ref):
  i = pl.program_id(0)
  A = a0_ref[...] + a1_ref[...]
  hin = x_ref[...] + A
  h1 = jnp.maximum(
      jnp.dot(hin, w1_ref[...], preferred_element_type=F32) + b1_ref[...], 0.0)
  h = jnp.dot(h1, w2_ref[...], preferred_element_type=F32) + b2_ref[...]
  logits = jnp.dot(h, ws_ref[...], preferred_element_type=F32) + bs_ref[...]
  m = jnp.max(logits, axis=-1, keepdims=True)
  e = jnp.exp(logits - m)
  S = e / jnp.sum(e, axis=-1, keepdims=True)
  rows = i * BN + lax.broadcasted_iota(I32, (BN, 1), 0)
  S = jnp.where(rows < N, S, 0.0)
  slo_ref[...] = S[:, 0:16]
  s1_ref[...] = S[:, 16:32]
  s2_ref[...] = S[:, 32:48]
  shi_ref[...] = S[:, 48:64]
  deg = A[:, 7:8]

  @pl.when(i == 0)
  def _():
    xp_ref[...] = jnp.zeros_like(xp_ref)
    sts_ref[...] = jnp.zeros_like(sts_ref)
    trd_ref[...] = jnp.zeros_like(trd_ref)

  cdims = (((0,), (0,)), ((), ()))
  xp_ref[...] += lax.dot_general(S, h, cdims, preferred_element_type=F32)
  sts_ref[...] += lax.dot_general(S, S, cdims, preferred_element_type=F32)
  trd_ref[...] += jnp.sum(deg * (S * S)).reshape(1, 1)


def _tc2_body(K, n_blocks, s0_ref, s1_ref, s2_ref, s3_ref,
              t0_ref, t1_ref, t2_ref, t3_ref, xp_ref, sts_ref,
              trd_ref, w3_ref, b3_ref, w4_ref, b4_ref, wl_ref, bl_ref,
              logp_ref, aux_ref, adj_acc):
  i = pl.program_id(0)

  @pl.when(i == 0)
  def _():
    adj_acc[...] = jnp.zeros_like(adj_acc)

  S64 = jnp.concatenate(
      [s0_ref[...], s1_ref[...], s2_ref[...], s3_ref[...]], axis=1)
  T64 = jnp.concatenate(
      [t0_ref[...], t1_ref[...], t2_ref[...], t3_ref[...]], axis=1)
  cdims = (((0,), (0,)), ((), ()))
  adj_acc[...] += lax.dot_general(T64, S64, cdims, preferred_element_type=F32)

  @pl.when(i == n_blocks - 1)
  def _():
    adj = adj_acc[...]
    ii = lax.broadcasted_iota(I32, (64, 64), 0)
    jj = lax.broadcasted_iota(I32, (64, 64), 1)
    diag = ii == jj
    cut_num = jnp.sum(jnp.where(diag, adj, 0.0))
    cut_loss = -cut_num / (trd_ref[0, 0] + 1e-9)
    sts = sts_ref[...]
    nrm = jnp.sqrt(jnp.sum(sts * sts))
    eye = jnp.where(diag & (ii < K), 1.0 / jnp.sqrt(jnp.float32(K)), 0.0)
    dmat = sts / (nrm + 1e-9) - eye
    ortho = jnp.sqrt(jnp.sum(dmat * dmat))
    aux_ref[...] = (cut_loss + ortho).reshape(1, 1)

    xp = xp_ref[...]
    hg = xp + jnp.dot(adj, xp, preferred_element_type=F32)
    hg = jnp.maximum(
        jnp.dot(hg, w3_ref[...], preferred_element_type=F32) + b3_ref[...], 0.0)
    hg = jnp.dot(hg, w4_ref[...], preferred_element_type=F32) + b4_ref[...]
    ri = lax.broadcasted_iota(I32, (64, 1), 0)
    g = jnp.sum(jnp.where(ri < K, hg, 0.0), axis=0, keepdims=True) / jnp.float32(K)
    out = jnp.dot(g, wl_ref[...], preferred_element_type=F32) + bl_ref[...]
    m = jnp.max(out, axis=-1, keepdims=True)
    lse = m + jnp.log(jnp.sum(jnp.exp(out - m), axis=-1, keepdims=True))
    logp_ref[...] = out - lse


def kernel(x, edge_index, edge_weight, batch, W1, b1, W2, b2, Ws, bs,
           W3, b3, W4, b4, Wl, bl):
  N, F = x.shape
  E = edge_index.shape[1]
  H = W1.shape[1]
  K = Ws.shape[1]
  C = Wl.shape[1]

  NP = ((N + 1 + 127) // 128) * 128
  EPQ = 128 * NS * G * NC * 2
  EP = ((E + EPQ - 1) // EPQ) * EPQ
  ER = EP // 128

  x16 = jnp.pad(x, ((0, NP - N), (0, 16 - F)))
  pad_e = EP - E
  srcp = jnp.concatenate(
      [edge_index[0], jnp.full((pad_e,), N, I32)]).reshape(ER, 128)
  dstp = jnp.concatenate(
      [edge_index[1], jnp.full((pad_e,), N, I32)]).reshape(ER, 128)
  wp = jnp.concatenate(
      [edge_weight, jnp.zeros((pad_e,), F32)]).reshape(ER, 128)

  a0, a1 = _sc_agg_kernel(NP, ER, ER // (NC * NS * G))(x16, srcp, dstp, wp)

  W1p = jnp.pad(W1, ((0, 16 - F), (0, 0)))
  Wsp = jnp.pad(Ws, ((0, 0), (0, 64 - K)))
  bsp = jnp.concatenate([bs, jnp.full((64 - K,), -1e30, F32)]).reshape(1, 64)

  BN = NP // 16
  n_blocks = NP // BN
  grid = (n_blocks,)
  s0, s1, s2, s3, xp64, sts, trd = pl.pallas_call(
      functools.partial(_tc1_body, N, BN),
      grid=grid,
      in_specs=[
          pl.BlockSpec((BN, 16), lambda i: (i, 0)),
          pl.BlockSpec((BN, 16), lambda i: (i, 0)),
          pl.BlockSpec((BN, 16), lambda i: (i, 0)),
          pl.BlockSpec((16, 64), lambda i: (0, 0)),
          pl.BlockSpec((1, 64), lambda i: (0, 0)),
          pl.BlockSpec((64, 64), lambda i: (0, 0)),
          pl.BlockSpec((1, 64), lambda i: (0, 0)),
          pl.BlockSpec((64, 64), lambda i: (0, 0)),
          pl.BlockSpec((1, 64), lambda i: (0, 0)),
      ],
      out_specs=[
          pl.BlockSpec((BN, 16), lambda i: (i, 0)),
          pl.BlockSpec((BN, 16), lambda i: (i, 0)),
          pl.BlockSpec((BN, 16), lambda i: (i, 0)),
          pl.BlockSpec((BN, 16), lambda i: (i, 0)),
          pl.BlockSpec((64, 64), lambda i: (0, 0)),
          pl.BlockSpec((64, 64), lambda i: (0, 0)),
          pl.BlockSpec((1, 1), lambda i: (0, 0)),
      ],
      out_shape=[
          jax.ShapeDtypeStruct((NP, 16), F32),
          jax.ShapeDtypeStruct((NP, 16), F32),
          jax.ShapeDtypeStruct((NP, 16), F32),
          jax.ShapeDtypeStruct((NP, 16), F32),
          jax.ShapeDtypeStruct((64, 64), F32),
          jax.ShapeDtypeStruct((64, 64), F32),
          jax.ShapeDtypeStruct((1, 1), F32),
      ],
  )(x16, a0, a1, W1p, b1.reshape(1, H), W2, b2.reshape(1, H),
    Wsp, bsp)

  t0, t1, t2, t3 = _sc_pool_kernel(NP, ER, ER // (NS * G))(
      s0, s1, s2, s3, srcp, dstp, wp)

  Wl128 = jnp.pad(Wl, ((0, 0), (0, 128 - C)))
  bl128 = jnp.concatenate([bl, jnp.full((128 - C,), -1e30, F32)]).reshape(1, 128)

  logp128, aux = pl.pallas_call(
      functools.partial(_tc2_body, K, n_blocks),
      grid=grid,
      in_specs=[
          pl.BlockSpec((BN, 16), lambda i: (i, 0)),
          pl.BlockSpec((BN, 16), lambda i: (i, 0)),
          pl.BlockSpec((BN, 16), lambda i: (i, 0)),
          pl.BlockSpec((BN, 16), lambda i: (i, 0)),
          pl.BlockSpec((BN, 16), lambda i: (i, 0)),
          pl.BlockSpec((BN, 16), lambda i: (i, 0)),
          pl.BlockSpec((BN, 16), lambda i: (i, 0)),
          pl.BlockSpec((BN, 16), lambda i: (i, 0)),
          pl.BlockSpec((64, 64), lambda i: (0, 0)),
          pl.BlockSpec((64, 64), lambda i: (0, 0)),
          pl.BlockSpec((1, 1), lambda i: (0, 0)),
          pl.BlockSpec((64, 64), lambda i: (0, 0)),
          pl.BlockSpec((1, 64), lambda i: (0, 0)),
          pl.BlockSpec((64, 64), lambda i: (0, 0)),
          pl.BlockSpec((1, 64), lambda i: (0, 0)),
          pl.BlockSpec((64, 128), lambda i: (0, 0)),
          pl.BlockSpec((1, 128), lambda i: (0, 0)),
      ],
      out_specs=[
          pl.BlockSpec((1, 128), lambda i: (0, 0)),
          pl.BlockSpec((1, 1), lambda i: (0, 0)),
      ],
      out_shape=[
          jax.ShapeDtypeStruct((1, 128), F32),
          jax.ShapeDtypeStruct((1, 1), F32),
      ],
      scratch_shapes=[pltpu.VMEM((64, 64), F32)],
  )(s0, s1, s2, s3, t0, t1, t2, t3, xp64, sts, trd, W3, b3.reshape(1, H),
    W4, b4.reshape(1, H), Wl128, bl128)

  return (logp128[0:1, 0:C], aux[0, 0])


# speedup vs baseline: 7.7456x; 1.1702x over previous
import functools

import jax
import jax.numpy as jnp
from jax import lax
from jax.experimental import pallas as pl
from jax.experimental.pallas import tpu as pltpu
from jax.experimental.pallas import tpu_sc as plsc

NC = 2
NS = 16
G = 10

F32 = jnp.float32
I32 = jnp.int32


def _edge_pipeline(tab_h, src_h, dst_h, w_h, acc_sh, bufs, base, n_iters,
                   transform, chunk):
  (idx_s, idx_d, w_v, rows_v, gsem, ssem) = bufs
  npairs = n_iters // 2

  def stage_and_fire(t, b):
    rb = base + t * G
    pltpu.sync_copy(src_h.at[pl.ds(rb, G)], idx_s[b])
    pltpu.sync_copy(dst_h.at[pl.ds(rb, G)], idx_d[b])
    pltpu.sync_copy(w_h.at[pl.ds(rb, G)], w_v[b])

    isb = idx_s[b]

    @plsc.parallel_loop(0, G * 8, 1, unroll=4)
    def _(g):
      v = isb[g >> 3, pl.ds((g & 7) * 16, 16)]
      isb[g >> 3, pl.ds((g & 7) * 16, 16)] = v * 8 + chunk

    for j in range(G):
      pltpu.async_copy(tab_h.at[idx_s[b].at[j]],
                       rows_v[b].at[pl.ds(j * 128, 128)], gsem[b])

  def wait_gathers(b):
    for j in range(G):
      pltpu.make_async_copy(tab_h.at[idx_s[b].at[j]],
                            rows_v[b].at[pl.ds(j * 128, 128)], gsem[b]).wait()

  def fire_scatters(b):
    for j in range(G):
      pltpu.async_copy(rows_v[b].at[pl.ds(j * 128, 128)],
                       acc_sh.at[idx_d[b].at[j]], ssem[b], add=True)

  def wait_scatters(b):
    for j in range(G):
      pltpu.make_async_copy(tab_h.at[idx_s[b].at[j]],
                            rows_v[b].at[pl.ds(j * 128, 128)], ssem[b]).wait()

  def sync_scatters(b):
    for j in range(G):
      pltpu.sync_copy(rows_v[b].at[pl.ds(j * 128, 128)],
                      acc_sh.at[idx_d[b].at[j]], add=True)

  stage_and_fire(0, 0)

  @pl.loop(0, npairs)
  def _(t2):
    wait_gathers(0)

    @pl.when(t2 > 0)
    def _():
      wait_scatters(1)

    stage_and_fire(2 * t2 + 1, 1)
    transform(0)
    fire_scatters(0)
    wait_gathers(1)
    wait_scatters(0)

    @pl.when(t2 < npairs - 1)
    def _():
      stage_and_fire(2 * t2 + 2, 0)

    transform(1)

    @pl.when(t2 < npairs - 1)
    def _():
      fire_scatters(1)

    @pl.when(t2 == npairs - 1)
    def _():
      sync_scatters(1)


def _zero_acc_slice(rows_v0, acc_sh, r0, rows_per_tile, width):
  @plsc.parallel_loop(0, G * 128, 1, unroll=8)
  def _(i):
    for h in range(width // 16):
      rows_v0[i, pl.ds(h * 16, 16)] = jnp.zeros((16,), F32)

  done = 0
  while done < rows_per_tile:
    n = min(G * 128, rows_per_tile - done)
    pltpu.sync_copy(rows_v0.at[pl.ds(0, n)], acc_sh.at[pl.ds(r0 + done, n)])
    done += n


def _sc_scratch(width):
  return [
      pltpu.VMEM((G, 128), I32), pltpu.VMEM((G, 128), I32),
      pltpu.VMEM((G, 128), I32), pltpu.VMEM((G, 128), I32),
      pltpu.VMEM((G, 128), F32), pltpu.VMEM((G, 128), F32),
      pltpu.VMEM((G * 128, width), F32),
      pltpu.VMEM((G * 128, width), F32),
      pltpu.SemaphoreType.DMA, pltpu.SemaphoreType.DMA,
      pltpu.SemaphoreType.DMA, pltpu.SemaphoreType.DMA,
  ]


def _sc_agg_kernel(NP, ER, n_iters):
  rows_per_tile = NP // NS

  mesh = plsc.VectorSubcoreMesh(
      core_axis_name="c", subcore_axis_name="s", num_cores=NC, num_subcores=NS)

  @functools.partial(
      pl.kernel,
      out_type=jax.ShapeDtypeStruct((NP, 128), F32),
      mesh=mesh,
      scratch_types=_sc_scratch(16)
      + [pltpu.VMEM_SHARED((NP, 16), F32)],
      compiler_params=pltpu.CompilerParams(use_tc_tiling_on_sc=False),
  )
  def k(x16_h, src_h, dst_h, w_h, out_h,
        is0, is1, id0, id1, wv0, wv1, rv0, rv1, gs0, gs1, ss0, ss1, acc_sh):
    c = lax.axis_index("c")
    s = lax.axis_index("s")
    r0 = s * rows_per_tile
    bufs = ((is0, is1), (id0, id1), (wv0, wv1), (rv0, rv1),
            (gs0, gs1), (ss0, ss1))

    _zero_acc_slice(rv0, acc_sh, r0, rows_per_tile, 16)
    plsc.subcore_barrier()

    def plant_w(b):
      rows_v, w_v = (rv0, rv1)[b], (wv0, wv1)[b]

      @plsc.parallel_loop(0, G * 8, 1, unroll=2)
      def _(g):
        w16 = w_v[g >> 3, pl.ds((g & 7) * 16, 16)]
        lane7 = lax.iota(I32, 16) == 7
        for m in range(16):
          r = g * 16 + m
          rows_v[r, :] = jnp.where(lane7, w16[m], rows_v[r, :])

    base = c * (ER // NC) + s * (ER // NC // NS)
    _edge_pipeline(x16_h, src_h, dst_h, w_h, acc_sh, bufs, base, n_iters,
                   plant_w, 0)
    plsc.subcore_barrier()

    @pl.when(c == 0)
    def _():
      pltpu.sync_copy(acc_sh.at[pl.ds(r0, rows_per_tile)],
                      out_h.at[pl.ds(r0, rows_per_tile), pl.ds(0, 16)])

    @pl.when(c == 1)
    def _():
      pltpu.sync_copy(acc_sh.at[pl.ds(r0, rows_per_tile)],
                      out_h.at[pl.ds(r0, rows_per_tile), pl.ds(16, 16)])

  return k


def _sc_pool_kernel(NP, ER, n_iters):
  rows_per_tile = NP // NS

  mesh = plsc.VectorSubcoreMesh(
      core_axis_name="c", subcore_axis_name="s", num_cores=NC, num_subcores=NS)

  @functools.partial(
      pl.kernel,
      out_type=jax.ShapeDtypeStruct((NP, 128), F32),
      mesh=mesh,
      scratch_types=_sc_scratch(16)
      + [pltpu.VMEM_SHARED((NP, 16), F32)],
      compiler_params=pltpu.CompilerParams(use_tc_tiling_on_sc=False),
  )
  def k(s8_h, src_h, dst_h, w_h, out_h,
        is0, is1, id0, id1, wv0, wv1, rv0, rv1, gs0, gs1, ss0, ss1, acc_sh):
    c = lax.axis_index("c")
    s = lax.axis_index("s")
    r0 = s * rows_per_tile
    bufs = ((is0, is1), (id0, id1), (wv0, wv1), (rv0, rv1),
            (gs0, gs1), (ss0, ss1))

    def scale_w(b):
      rows_v, w_v = (rv0, rv1)[b], (wv0, wv1)[b]

      @plsc.parallel_loop(0, G * 8, 1, unroll=2)
      def _(g):
        w16 = w_v[g >> 3, pl.ds((g & 7) * 16, 16)]
        for m in range(16):
          r = g * 16 + m
          rows_v[r, :] = rows_v[r, :] * w16[m]

    def one_chunk(chunk):
      _zero_acc_slice(rv0, acc_sh, r0, rows_per_tile, 16)
      plsc.subcore_barrier()
      base = s * (ER // NS)
      _edge_pipeline(s8_h, src_h, dst_h, w_h, acc_sh, bufs, base, n_iters,
                     scale_w, chunk)
      plsc.subcore_barrier()
      pltpu.sync_copy(
          acc_sh.at[pl.ds(r0, rows_per_tile)],
          out_h.at[pl.ds(r0, rows_per_tile), pl.ds(chunk * 16, 16)])
      plsc.subcore_barrier()

    @pl.when(c == 0)
    def _():
      one_chunk(0)
      one_chunk(1)

    @pl.when(c == 1)
    def _():
      one_chunk(2)
      one_chunk(3)

  return k


def _tc1_body(N, BN, x_ref, a_ref, w1_ref, b1_ref, w2_ref, b2_ref,
              ws_ref, bs_ref, s_ref, xp_ref, sts_ref, trd_ref):
  i = pl.program_id(0)
  A = a_ref[:, 0:16] + a_ref[:, 16:32]
  hin = x_ref[:, 0:16] + A
  h1 = jnp.maximum(
      jnp.dot(hin, w1_ref[...], preferred_element_type=F32) + b1_ref[...], 0.0)
  h = jnp.dot(h1, w2_ref[...], preferred_element_type=F32) + b2_ref[...]
  logits = jnp.dot(h, ws_ref[...], preferred_element_type=F32) + bs_ref[...]
  m = jnp.max(logits, axis=-1, keepdims=True)
  e = jnp.exp(logits - m)
  S = e / jnp.sum(e, axis=-1, keepdims=True)
  rows = i * BN + lax.broadcasted_iota(I32, (BN, 1), 0)
  S = jnp.where(rows < N, S, 0.0)
  s_ref[:, 0:64] = S
  s_ref[:, 64:128] = jnp.zeros((BN, 64), F32)
  deg = A[:, 7:8]

  @pl.when(i == 0)
  def _():
    xp_ref[...] = jnp.zeros_like(xp_ref)
    sts_ref[...] = jnp.zeros_like(sts_ref)
    trd_ref[...] = jnp.zeros_like(trd_ref)

  cdims = (((0,), (0,)), ((), ()))
  xp_ref[...] += lax.dot_general(S, h, cdims, preferred_element_type=F32)
  sts_ref[...] += lax.dot_general(S, S, cdims, preferred_element_type=F32)
  trd_ref[...] += jnp.sum(deg * (S * S)).reshape(1, 1)


def _tc2_body(K, n_blocks, s_ref, t_ref, xp_ref, sts_ref,
              trd_ref, w3_ref, b3_ref, w4_ref, b4_ref, wl_ref, bl_ref,
              logp_ref, aux_ref, adj_acc):
  i = pl.program_id(0)

  @pl.when(i == 0)
  def _():
    adj_acc[...] = jnp.zeros_like(adj_acc)

  S64 = s_ref[:, 0:64]
  T64 = t_ref[:, 0:64]
  cdims = (((0,), (0,)), ((), ()))
  adj_acc[...] += lax.dot_general(T64, S64, cdims, preferred_element_type=F32)

  @pl.when(i == n_blocks - 1)
  def _():
    adj = adj_acc[...]
    ii = lax.broadcasted_iota(I32, (64, 64), 0)
    jj = lax.broadcasted_iota(I32, (64, 64), 1)
    diag = ii == jj
    cut_num = jnp.sum(jnp.where(diag, adj, 0.0))
    cut_loss = -cut_num / (trd_ref[0, 0] + 1e-9)
    sts = sts_ref[...]
    nrm = jnp.sqrt(jnp.sum(sts * sts))
    eye = jnp.where(diag & (ii < K), 1.0 / jnp.sqrt(jnp.float32(K)), 0.0)
    dmat = sts / (nrm + 1e-9) - eye
    ortho = jnp.sqrt(jnp.sum(dmat * dmat))
    aux_ref[...] = (cut_loss + ortho).reshape(1, 1)

    xp = xp_ref[...]
    hg = xp + jnp.dot(adj, xp, preferred_element_type=F32)
    hg = jnp.maximum(
        jnp.dot(hg, w3_ref[...], preferred_element_type=F32) + b3_ref[...], 0.0)
    hg = jnp.dot(hg, w4_ref[...], preferred_element_type=F32) + b4_ref[...]
    ri = lax.broadcasted_iota(I32, (64, 1), 0)
    g = jnp.sum(jnp.where(ri < K, hg, 0.0), axis=0, keepdims=True) / jnp.float32(K)
    out = jnp.dot(g, wl_ref[...], preferred_element_type=F32) + bl_ref[...]
    m = jnp.max(out, axis=-1, keepdims=True)
    lse = m + jnp.log(jnp.sum(jnp.exp(out - m), axis=-1, keepdims=True))
    logp_ref[...] = out - lse


def kernel(x, edge_index, edge_weight, batch, W1, b1, W2, b2, Ws, bs,
           W3, b3, W4, b4, Wl, bl):
  N, F = x.shape
  E = edge_index.shape[1]
  H = W1.shape[1]
  K = Ws.shape[1]
  C = Wl.shape[1]

  NP = ((N + 1 + 127) // 128) * 128
  EPQ = 128 * NS * G * NC * 2
  EP = ((E + EPQ - 1) // EPQ) * EPQ
  ER = EP // 128

  x128 = jnp.pad(x, ((0, NP - N), (0, 128 - F)))
  x8v = x128.reshape(8 * NP, 16)
  pad_e = EP - E
  srcp = jnp.concatenate(
      [edge_index[0], jnp.full((pad_e,), N, I32)]).reshape(ER, 128)
  dstp = jnp.concatenate(
      [edge_index[1], jnp.full((pad_e,), N, I32)]).reshape(ER, 128)
  wp = jnp.concatenate(
      [edge_weight, jnp.zeros((pad_e,), F32)]).reshape(ER, 128)

  a128 = _sc_agg_kernel(NP, ER, ER // (NC * NS * G))(x8v, srcp, dstp, wp)

  W1p = jnp.pad(W1, ((0, 16 - F), (0, 0)))
  Wsp = jnp.pad(Ws, ((0, 0), (0, 64 - K)))
  bsp = jnp.concatenate([bs, jnp.full((64 - K,), -1e30, F32)]).reshape(1, 64)

  BN = NP // 16
  n_blocks = NP // BN
  grid = (n_blocks,)
  S128, xp64, sts, trd = pl.pallas_call(
      functools.partial(_tc1_body, N, BN),
      grid=grid,
      in_specs=[
          pl.BlockSpec((BN, 128), lambda i: (i, 0)),
          pl.BlockSpec((BN, 128), lambda i: (i, 0)),
          pl.BlockSpec((16, 64), lambda i: (0, 0)),
          pl.BlockSpec((1, 64), lambda i: (0, 0)),
          pl.BlockSpec((64, 64), lambda i: (0, 0)),
          pl.BlockSpec((1, 64), lambda i: (0, 0)),
          pl.BlockSpec((64, 64), lambda i: (0, 0)),
          pl.BlockSpec((1, 64), lambda i: (0, 0)),
      ],
      out_specs=[
          pl.BlockSpec((BN, 128), lambda i: (i, 0)),
          pl.BlockSpec((64, 64), lambda i: (0, 0)),
          pl.BlockSpec((64, 64), lambda i: (0, 0)),
          pl.BlockSpec((1, 1), lambda i: (0, 0)),
      ],
      out_shape=[
          jax.ShapeDtypeStruct((NP, 128), F32),
          jax.ShapeDtypeStruct((64, 64), F32),
          jax.ShapeDtypeStruct((64, 64), F32),
          jax.ShapeDtypeStruct((1, 1), F32),
      ],
  )(x128, a128, W1p, b1.reshape(1, H), W2, b2.reshape(1, H),
    Wsp, bsp)

  s8v = S128.reshape(8 * NP, 16)
  T128 = _sc_pool_kernel(NP, ER, ER // (NS * G))(s8v, srcp, dstp, wp)

  Wl128 = jnp.pad(Wl, ((0, 0), (0, 128 - C)))
  bl128 = jnp.concatenate([bl, jnp.full((128 - C,), -1e30, F32)]).reshape(1, 128)

  logp128, aux = pl.pallas_call(
      functools.partial(_tc2_body, K, n_blocks),
      grid=grid,
      in_specs=[
          pl.BlockSpec((BN, 128), lambda i: (i, 0)),
          pl.BlockSpec((BN, 128), lambda i: (i, 0)),
          pl.BlockSpec((64, 64), lambda i: (0, 0)),
          pl.BlockSpec((64, 64), lambda i: (0, 0)),
          pl.BlockSpec((1, 1), lambda i: (0, 0)),
          pl.BlockSpec((64, 64), lambda i: (0, 0)),
          pl.BlockSpec((1, 64), lambda i: (0, 0)),
          pl.BlockSpec((64, 64), lambda i: (0, 0)),
          pl.BlockSpec((1, 64), lambda i: (0, 0)),
          pl.BlockSpec((64, 128), lambda i: (0, 0)),
          pl.BlockSpec((1, 128), lambda i: (0, 0)),
      ],
      out_specs=[
          pl.BlockSpec((1, 128), lambda i: (0, 0)),
          pl.BlockSpec((1, 1), lambda i: (0, 0)),
      ],
      out_shape=[
          jax.ShapeDtypeStruct((1, 128), F32),
          jax.ShapeDtypeStruct((1, 1), F32),
      ],
      scratch_shapes=[pltpu.VMEM((64, 64), F32)],
  )(S128, T128, xp64, sts, trd, W3, b3.reshape(1, H),
    W4, b4.reshape(1, H), Wl128, bl128)

  return (logp128[0:1, 0:C], aux[0, 0])
